# Initial kernel scaffold; baseline (speedup 1.0000x reference)
import jax
import jax.numpy as jnp
from jax.experimental import pallas as pl


def kernel(users, movies, user_neighbors, movie_neighbors, input_ids, attention_mask, node_emb, relation_emb, att_W1, att_b1, att_W2, att_b2, Wu1, bu1, Wu2, bu2, Wv1, bv1, Wv2, bv2):
    raise NotImplementedError("write your pallas kernel here")


# speedup vs baseline: 26.9231x; 26.9231x over previous
import functools

import jax
import jax.numpy as jnp
from jax import lax
from jax.experimental import pallas as pl
from jax.experimental.pallas import tpu as pltpu
from jax.experimental.pallas import tpu_sc as plsc

NC, NS = 2, 16
NW = NC * NS
DIM = 32
NF = 4
ROW = NF * DIM
CHUNK = 128


def _sc_gather(idx_h, idx_r, idx_t, idx_um, node_flat, rel_flat):
    NB = idx_h.shape[0]
    per_w = NB // NW
    n_chunks = per_w // CHUNK
    UM = idx_um.shape[0]
    um_per_w = UM // NW

    mesh = plsc.VectorSubcoreMesh(core_axis_name="c", subcore_axis_name="s")

    @functools.partial(
        pl.kernel,
        out_type=(
            jax.ShapeDtypeStruct((NB, ROW), jnp.float32),
            jax.ShapeDtypeStruct((NB, ROW), jnp.float32),
            jax.ShapeDtypeStruct((NB, ROW), jnp.float32),
            jax.ShapeDtypeStruct((UM, ROW), jnp.float32),
        ),
        mesh=mesh,
        scratch_types=[
            pltpu.VMEM((CHUNK,), jnp.int32),
            pltpu.VMEM((CHUNK, ROW), jnp.float32),
            pltpu.VMEM((um_per_w,), jnp.int32),
            pltpu.VMEM((um_per_w, ROW), jnp.float32),
            pltpu.SemaphoreType.DMA,
        ],
    )
    def gather_kernel(idx_h_hbm, idx_r_hbm, idx_t_hbm, idx_um_hbm,
                      node_hbm, rel_hbm,
                      out_h, out_r, out_t, out_um,
                      idx_v, rows_v, idx_um_v, rows_um_v, gsem):
        wid = lax.axis_index("s") * NC + lax.axis_index("c")
        base = wid * per_w

        def run(idx_hbm, tab_hbm, out_hbm):
            def body(i, carry):
                off = pl.multiple_of(base + i * CHUNK, CHUNK)
                pltpu.sync_copy(idx_hbm.at[pl.ds(off, CHUNK)], idx_v)
                pltpu.async_copy(tab_hbm.at[idx_v], rows_v, gsem).wait()
                pltpu.sync_copy(rows_v, out_hbm.at[pl.ds(off, CHUNK)])
                return carry
            lax.fori_loop(0, n_chunks, body, 0)

        run(idx_h_hbm, node_hbm, out_h)
        run(idx_t_hbm, node_hbm, out_t)
        run(idx_r_hbm, rel_hbm, out_r)

        ub = pl.multiple_of(wid * um_per_w, 8)
        pltpu.sync_copy(idx_um_hbm.at[pl.ds(ub, um_per_w)], idx_um_v)
        pltpu.async_copy(node_hbm.at[idx_um_v], rows_um_v, gsem).wait()
        pltpu.sync_copy(rows_um_v, out_um.at[pl.ds(ub, um_per_w)])

    return gather_kernel(idx_h, idx_r, idx_t, idx_um, node_flat, rel_flat)


def _tc_compute(gh, gr, gt, gum, Wcat, b1t, P, E, R, G, B, NM):
    UB = 128
    RPB = UB * NM
    n_ub = B // UB
    n_steps = G * n_ub

    def body(gh_ref, gr_ref, gt_ref, gum_ref, Wc_ref, b1_ref, P_ref, E_ref,
             R_ref, out_ref, acc_u, acc_v):
        g = pl.program_id(0)
        u = pl.program_id(1)
        step = g * n_ub + u

        @pl.when(step == 0)
        def _init():
            acc_u[...] = gum_ref[0:B, :]
            acc_v[...] = gum_ref[B:2 * B, :]

        x = jnp.concatenate([gh_ref[...], gr_ref[...]], axis=1)
        H = jnp.dot(x, Wc_ref[...], preferred_element_type=jnp.float32,
                    precision=lax.Precision.HIGHEST)
        H = jnp.maximum(H + b1_ref[...], 0.0)
        att = jnp.dot(H, P_ref[...], preferred_element_type=jnp.float32,
                      precision=lax.Precision.HIGHEST)
        att3 = att.reshape(UB, NM, ROW)
        mx = jnp.max(att3, axis=1, keepdims=True)
        eatt = jnp.exp(att3 - mx).reshape(RPB, ROW)
        eb = jnp.dot(eatt, E_ref[...], preferred_element_type=jnp.float32,
                     precision=lax.Precision.HIGHEST)
        numer = (eb * gt_ref[...]).reshape(UB, NM, ROW).sum(axis=1)
        denom = eb.reshape(UB, NM, ROW).sum(axis=1)
        gout = numer / denom

        rowbase = u * UB

        @pl.when(g < 2)
        def _acc_user():
            acc_u[pl.ds(rowbase, UB), :] += gout

        @pl.when(g >= 2)
        def _acc_item():
            acc_v[pl.ds(rowbase, UB), :] += gout

        @pl.when(step == n_steps - 1)
        def _final():
            euf = jnp.dot(acc_u[...], R_ref[...],
                          preferred_element_type=jnp.float32,
                          precision=lax.Precision.HIGHEST)
            evf = jnp.dot(acc_v[...], R_ref[...],
                          preferred_element_type=jnp.float32,
                          precision=lax.Precision.HIGHEST)
            s = jnp.sum(euf * evf, axis=1, keepdims=True)
            out_ref[...] = jnp.broadcast_to(jax.nn.sigmoid(s), (B, ROW))

    return pl.pallas_call(
        body,
        grid=(G, n_ub),
        in_specs=[
            pl.BlockSpec((RPB, ROW), lambda g, u: (g * n_ub + u, 0)),
            pl.BlockSpec((RPB, ROW), lambda g, u: (g * n_ub + u, 0)),
            pl.BlockSpec((RPB, ROW), lambda g, u: (g * n_ub + u, 0)),
            pl.BlockSpec((2 * B, ROW), lambda g, u: (0, 0)),
            pl.BlockSpec((2 * ROW, ROW), lambda g, u: (0, 0)),
            pl.BlockSpec((1, ROW), lambda g, u: (0, 0)),
            pl.BlockSpec((ROW, ROW), lambda g, u: (0, 0)),
            pl.BlockSpec((ROW, ROW), lambda g, u: (0, 0)),
            pl.BlockSpec((ROW, ROW), lambda g, u: (0, 0)),
        ],
        out_specs=pl.BlockSpec((B, ROW), lambda g, u: (0, 0)),
        out_shape=jax.ShapeDtypeStruct((B, ROW), jnp.float32),
        scratch_shapes=[
            pltpu.VMEM((B, ROW), jnp.float32),
            pltpu.VMEM((B, ROW), jnp.float32),
        ],
    )(gh, gr, gt, gum, Wcat, b1t, P, E, R)


def kernel(users, movies, user_neighbors, movie_neighbors, input_ids,
           attention_mask, node_emb, relation_emb, att_W1, att_b1, att_W2,
           att_b2, Wu1, bu1, Wu2, bu2, Wv1, bv1, Wv2, bv2):
    del input_ids, attention_mask
    del Wu1, bu1, Wu2, bu2, Wv1, bv1, Wv2, bv2
    del att_b2

    B = users.shape[0]
    NM = user_neighbors.shape[3]
    NL = user_neighbors.shape[1]
    G = 2 * NL

    node_flat = node_emb.reshape(node_emb.shape[0], ROW)
    rel_flat = relation_emb.reshape(relation_emb.shape[0], ROW)
    idx_h = jnp.concatenate(
        [user_neighbors[0].reshape(-1), movie_neighbors[0].reshape(-1)])
    idx_r = jnp.concatenate(
        [user_neighbors[1].reshape(-1), movie_neighbors[1].reshape(-1)])
    idx_t = jnp.concatenate(
        [user_neighbors[2].reshape(-1), movie_neighbors[2].reshape(-1)])
    idx_um = jnp.concatenate([users, movies])

    eye4 = jnp.eye(NF, dtype=jnp.float32)
    W1h = att_W1[:DIM, :]
    W1r = att_W1[DIM:, :]
    Wcat = jnp.concatenate(
        [jnp.kron(eye4, W1h), jnp.kron(eye4, W1r)], axis=0)
    b1t = jnp.tile(att_b1, NF)[None, :]
    P = jnp.pad(jnp.kron(eye4, att_W2), ((0, 0), (0, ROW - NF)))
    E = jnp.pad(jnp.kron(eye4, jnp.ones((1, DIM), jnp.float32)),
                ((0, ROW - NF), (0, 0)))
    R = jnp.pad(jnp.kron(jnp.ones((NF, 1), jnp.float32),
                         jnp.eye(DIM, dtype=jnp.float32)),
                ((0, 0), (0, ROW - DIM)))

    gh, gr, gt, gum = _sc_gather(idx_h, idx_r, idx_t, idx_um,
                                 node_flat, rel_flat)
    out = _tc_compute(gh, gr, gt, gum, Wcat, b1t, P, E, R, G, B, NM)
    return out[:, 0]


# speedup vs baseline: 46.7924x; 1.7380x over previous
import functools

import jax
import jax.numpy as jnp
from jax import lax
from jax.experimental import pallas as pl
from jax.experimental.pallas import tpu as pltpu
from jax.experimental.pallas import tpu_sc as plsc

NC, NS = 2, 16
NW = NC * NS
DIM = 32
NF = 4
ROW = NF * DIM
CHUNK = 128


def _sc_gather(idx_h, idx_r, idx_t, idx_um, node_flat, rel_flat):
    NB = idx_h.shape[0]
    per_w = NB // NW
    n_chunks = per_w // CHUNK
    UM = idx_um.shape[0]
    um_per_w = UM // NW

    mesh = plsc.VectorSubcoreMesh(core_axis_name="c", subcore_axis_name="s")

    @functools.partial(
        pl.kernel,
        out_type=(
            jax.ShapeDtypeStruct((NB, ROW), jnp.float32),
            jax.ShapeDtypeStruct((NB, ROW), jnp.float32),
            jax.ShapeDtypeStruct((NB, ROW), jnp.float32),
            jax.ShapeDtypeStruct((UM, ROW), jnp.float32),
        ),
        mesh=mesh,
        scratch_types=[
            pltpu.VMEM((CHUNK,), jnp.int32),
            pltpu.VMEM((CHUNK, ROW), jnp.float32),
            pltpu.VMEM((um_per_w,), jnp.int32),
            pltpu.VMEM((um_per_w, ROW), jnp.float32),
            pltpu.SemaphoreType.DMA,
        ],
    )
    def gather_kernel(idx_h_hbm, idx_r_hbm, idx_t_hbm, idx_um_hbm,
                      node_hbm, rel_hbm,
                      out_h, out_r, out_t, out_um,
                      idx_v, rows_v, idx_um_v, rows_um_v, gsem):
        wid = lax.axis_index("s") * NC + lax.axis_index("c")
        base = wid * per_w

        def run(idx_hbm, tab_hbm, out_hbm):
            def body(i, carry):
                off = pl.multiple_of(base + i * CHUNK, CHUNK)
                pltpu.sync_copy(idx_hbm.at[pl.ds(off, CHUNK)], idx_v)
                pltpu.async_copy(tab_hbm.at[idx_v], rows_v, gsem).wait()
                pltpu.sync_copy(rows_v, out_hbm.at[pl.ds(off, CHUNK)])
                return carry
            lax.fori_loop(0, n_chunks, body, 0)

        run(idx_h_hbm, node_hbm, out_h)
        run(idx_t_hbm, node_hbm, out_t)
        run(idx_r_hbm, rel_hbm, out_r)

        ub = pl.multiple_of(wid * um_per_w, 8)
        pltpu.sync_copy(idx_um_hbm.at[pl.ds(ub, um_per_w)], idx_um_v)
        pltpu.async_copy(node_hbm.at[idx_um_v], rows_um_v, gsem).wait()
        pltpu.sync_copy(rows_um_v, out_um.at[pl.ds(ub, um_per_w)])

    return gather_kernel(idx_h, idx_r, idx_t, idx_um, node_flat, rel_flat)


def _tc_compute(gh, gr, gt, gum, Wh, Wr, b1t, w2t, Bones, R, G, B, NM):
    UB = 128
    RPB = UB * NM
    n_ub = B // UB
    n_steps = G * n_ub

    def body(gh_ref, gr_ref, gt_ref, gum_ref, Wh_ref, Wr_ref, b1_ref, w2_ref,
             Bo_ref, R_ref, out_ref, acc_u, acc_v):
        g = pl.program_id(0)
        u = pl.program_id(1)
        step = g * n_ub + u

        @pl.when(step == 0)
        def _init():
            acc_u[...] = gum_ref[0:B, :]
            acc_v[...] = gum_ref[B:2 * B, :]

        H = (jnp.dot(gh_ref[...], Wh_ref[...],
                     preferred_element_type=jnp.float32,
                     precision=lax.Precision.DEFAULT)
             + jnp.dot(gr_ref[...], Wr_ref[...],
                       preferred_element_type=jnp.float32,
                       precision=lax.Precision.DEFAULT))
        H = jnp.maximum(H + b1_ref[...], 0.0)
        S = H * w2_ref[...]
        attb = jnp.dot(S, Bo_ref[...], preferred_element_type=jnp.float32,
                       precision=lax.Precision.DEFAULT)
        att3 = attb.reshape(UB, NM, ROW)
        mx = jnp.max(att3, axis=1, keepdims=True)
        eb = jnp.exp(att3 - mx).reshape(RPB, ROW)
        numer = (eb * gt_ref[...]).reshape(UB, NM, ROW).sum(axis=1)
        denom = eb.reshape(UB, NM, ROW).sum(axis=1)
        gout = numer / denom

        rowbase = u * UB

        @pl.when(g < 2)
        def _acc_user():
            acc_u[pl.ds(rowbase, UB), :] += gout

        @pl.when(g >= 2)
        def _acc_item():
            acc_v[pl.ds(rowbase, UB), :] += gout

        @pl.when(step == n_steps - 1)
        def _final():
            euf = jnp.dot(acc_u[...], R_ref[...],
                          preferred_element_type=jnp.float32,
                          precision=lax.Precision.HIGHEST)
            evf = jnp.dot(acc_v[...], R_ref[...],
                          preferred_element_type=jnp.float32,
                          precision=lax.Precision.HIGHEST)
            s = jnp.sum(euf * evf, axis=1, keepdims=True)
            out_ref[...] = jnp.broadcast_to(jax.nn.sigmoid(s), (B, ROW))

    return pl.pallas_call(
        body,
        grid=(G, n_ub),
        in_specs=[
            pl.BlockSpec((RPB, ROW), lambda g, u: (g * n_ub + u, 0)),
            pl.BlockSpec((RPB, ROW), lambda g, u: (g * n_ub + u, 0)),
            pl.BlockSpec((RPB, ROW), lambda g, u: (g * n_ub + u, 0)),
            pl.BlockSpec((2 * B, ROW), lambda g, u: (0, 0)),
            pl.BlockSpec((ROW, ROW), lambda g, u: (0, 0)),
            pl.BlockSpec((ROW, ROW), lambda g, u: (0, 0)),
            pl.BlockSpec((1, ROW), lambda g, u: (0, 0)),
            pl.BlockSpec((1, ROW), lambda g, u: (0, 0)),
            pl.BlockSpec((ROW, ROW), lambda g, u: (0, 0)),
            pl.BlockSpec((ROW, ROW), lambda g, u: (0, 0)),
        ],
        out_specs=pl.BlockSpec((B, ROW), lambda g, u: (0, 0)),
        out_shape=jax.ShapeDtypeStruct((B, ROW), jnp.float32),
        scratch_shapes=[
            pltpu.VMEM((B, ROW), jnp.float32),
            pltpu.VMEM((B, ROW), jnp.float32),
        ],
    )(gh, gr, gt, gum, Wh, Wr, b1t, w2t, Bones, R)


def kernel(users, movies, user_neighbors, movie_neighbors, input_ids,
           attention_mask, node_emb, relation_emb, att_W1, att_b1, att_W2,
           att_b2, Wu1, bu1, Wu2, bu2, Wv1, bv1, Wv2, bv2):
    del input_ids, attention_mask
    del Wu1, bu1, Wu2, bu2, Wv1, bv1, Wv2, bv2
    del att_b2

    B = users.shape[0]
    NM = user_neighbors.shape[3]
    NL = user_neighbors.shape[1]
    G = 2 * NL

    node_flat = node_emb.reshape(node_emb.shape[0], ROW)
    rel_flat = relation_emb.reshape(relation_emb.shape[0], ROW)
    idx_h = jnp.concatenate(
        [user_neighbors[0].reshape(-1), movie_neighbors[0].reshape(-1)])
    idx_r = jnp.concatenate(
        [user_neighbors[1].reshape(-1), movie_neighbors[1].reshape(-1)])
    idx_t = jnp.concatenate(
        [user_neighbors[2].reshape(-1), movie_neighbors[2].reshape(-1)])
    idx_um = jnp.concatenate([users, movies])

    eye4 = jnp.eye(NF, dtype=jnp.float32)
    Wh = jnp.kron(eye4, att_W1[:DIM, :])
    Wr = jnp.kron(eye4, att_W1[DIM:, :])
    b1t = jnp.tile(att_b1, NF)[None, :]
    w2t = jnp.tile(att_W2[:, 0], NF)[None, :]
    Bones = jnp.kron(eye4, jnp.ones((DIM, DIM), jnp.float32))
    R = jnp.pad(jnp.kron(jnp.ones((NF, 1), jnp.float32),
                         jnp.eye(DIM, dtype=jnp.float32)),
                ((0, 0), (0, ROW - DIM)))

    gh, gr, gt, gum = _sc_gather(idx_h, idx_r, idx_t, idx_um,
                                 node_flat, rel_flat)
    out = _tc_compute(gh, gr, gt, gum, Wh, Wr, b1t, w2t, Bones, R, G, B, NM)
    return out[:, 0]


# speedup vs baseline: 62.3606x; 1.3327x over previous
import functools

import jax
import jax.numpy as jnp
from jax import lax
from jax.experimental import pallas as pl
from jax.experimental.pallas import tpu as pltpu
from jax.experimental.pallas import tpu_sc as plsc

NC, NS = 2, 16
NW = NC * NS
DIM = 32
NF = 4
ROW = NF * DIM
CHUNK = 128


def _sc_gather(idx_h, idx_r, idx_t, idx_um, node_flat, rel_flat):
    NB = idx_h.shape[0]
    per_w = NB // NW
    n_chunks = per_w // CHUNK
    UM = idx_um.shape[0]
    um_per_w = UM // NW

    mesh = plsc.VectorSubcoreMesh(core_axis_name="c", subcore_axis_name="s")

    idx_h2 = idx_h.reshape(-1, CHUNK)
    idx_r2 = idx_r.reshape(-1, CHUNK)
    idx_t2 = idx_t.reshape(-1, CHUNK)
    idx_um2 = idx_um.reshape(NW, um_per_w)

    @functools.partial(
        pl.kernel,
        out_type=(
            jax.ShapeDtypeStruct((NB, ROW), jnp.float32),
            jax.ShapeDtypeStruct((NB, ROW), jnp.float32),
            jax.ShapeDtypeStruct((NB, ROW), jnp.float32),
            jax.ShapeDtypeStruct((UM, ROW), jnp.float32),
        ),
        mesh=mesh,
        scratch_types=[
            pltpu.VMEM((n_chunks, CHUNK), jnp.int32),
            pltpu.VMEM((CHUNK, ROW), jnp.float32),
            pltpu.VMEM((CHUNK, ROW), jnp.float32),
            pltpu.VMEM((um_per_w,), jnp.int32),
            pltpu.VMEM((um_per_w, ROW), jnp.float32),
            pltpu.SemaphoreType.DMA,
            pltpu.SemaphoreType.DMA,
            pltpu.SemaphoreType.DMA,
            pltpu.SemaphoreType.DMA,
        ],
    )
    def gather_kernel(idx_h_hbm, idx_r_hbm, idx_t_hbm, idx_um_hbm,
                      node_hbm, rel_hbm,
                      out_h, out_r, out_t, out_um,
                      idx_all, rows0, rows1, idx_um_v, rows_um_v,
                      gsem0, gsem1, osem0, osem1):
        wid = lax.axis_index("s") * NC + lax.axis_index("c")
        base = wid * per_w
        rows = (rows0, rows1)
        gsem = (gsem0, gsem1)
        osem = (osem0, osem1)

        def run(idx_hbm, tab_hbm, out_hbm):
            pltpu.sync_copy(idx_hbm.at[pl.ds(wid * n_chunks, n_chunks)],
                            idx_all)
            for b in range(2):
                pltpu.async_copy(tab_hbm.at[idx_all.at[b]], rows[b], gsem[b])

            def body(j, carry):
                for b in range(2):
                    i = 2 * j + b
                    off = pl.multiple_of(base + i * CHUNK, CHUNK)
                    pltpu.make_async_copy(tab_hbm.at[idx_all.at[i]],
                                          rows[b], gsem[b]).wait()
                    pltpu.async_copy(rows[b], out_hbm.at[pl.ds(off, CHUNK)],
                                     osem[b])

                    @pl.when(i + 2 < n_chunks)
                    def _next():
                        pltpu.make_async_copy(
                            rows[b], out_hbm.at[pl.ds(off, CHUNK)],
                            osem[b]).wait()
                        pltpu.async_copy(tab_hbm.at[idx_all.at[i + 2]],
                                         rows[b], gsem[b])
                return carry
            lax.fori_loop(0, n_chunks // 2, body, 0)
            for b in range(2):
                pltpu.make_async_copy(rows[b], out_hbm.at[pl.ds(base, CHUNK)],
                                      osem[b]).wait()

        run(idx_h_hbm, node_hbm, out_h)
        run(idx_t_hbm, node_hbm, out_t)
        run(idx_r_hbm, rel_hbm, out_r)

        ub = pl.multiple_of(wid * um_per_w, 8)
        pltpu.sync_copy(idx_um_hbm.at[wid], idx_um_v)
        pltpu.async_copy(node_hbm.at[idx_um_v], rows_um_v, gsem0).wait()
        pltpu.sync_copy(rows_um_v, out_um.at[pl.ds(ub, um_per_w)])

    return gather_kernel(idx_h2, idx_r2, idx_t2, idx_um2, node_flat, rel_flat)


def _tc_compute(gh, gr, gt, gum, Wh, Wr, b1t, w2t, Bones, R, G, B, NM):
    UB = 128
    RPB = UB * NM
    n_ub = B // UB
    n_steps = G * n_ub

    def body(gh_ref, gr_ref, gt_ref, gum_ref, Wh_ref, Wr_ref, b1_ref, w2_ref,
             Bo_ref, R_ref, out_ref, acc_u, acc_v):
        g = pl.program_id(0)
        u = pl.program_id(1)
        step = g * n_ub + u

        @pl.when(step == 0)
        def _init():
            acc_u[...] = gum_ref[0:B, :]
            acc_v[...] = gum_ref[B:2 * B, :]

        H = (jnp.dot(gh_ref[...], Wh_ref[...],
                     preferred_element_type=jnp.float32,
                     precision=lax.Precision.DEFAULT)
             + jnp.dot(gr_ref[...], Wr_ref[...],
                       preferred_element_type=jnp.float32,
                       precision=lax.Precision.DEFAULT))
        H = jnp.maximum(H + b1_ref[...], 0.0)
        S = H * w2_ref[...]
        attb = jnp.dot(S, Bo_ref[...], preferred_element_type=jnp.float32,
                       precision=lax.Precision.DEFAULT)
        att3 = attb.reshape(UB, NM, ROW)
        mx = jnp.max(att3, axis=1, keepdims=True)
        eb = jnp.exp(att3 - mx).reshape(RPB, ROW)
        numer = (eb * gt_ref[...]).reshape(UB, NM, ROW).sum(axis=1)
        denom = eb.reshape(UB, NM, ROW).sum(axis=1)
        gout = numer / denom

        rowbase = u * UB

        @pl.when(g < 2)
        def _acc_user():
            acc_u[pl.ds(rowbase, UB), :] += gout

        @pl.when(g >= 2)
        def _acc_item():
            acc_v[pl.ds(rowbase, UB), :] += gout

        @pl.when(step == n_steps - 1)
        def _final():
            euf = jnp.dot(acc_u[...], R_ref[...],
                          preferred_element_type=jnp.float32,
                          precision=lax.Precision.HIGHEST)
            evf = jnp.dot(acc_v[...], R_ref[...],
                          preferred_element_type=jnp.float32,
                          precision=lax.Precision.HIGHEST)
            s = jnp.sum(euf * evf, axis=1, keepdims=True)
            out_ref[...] = jnp.broadcast_to(jax.nn.sigmoid(s), (B, ROW))

    return pl.pallas_call(
        body,
        grid=(G, n_ub),
        in_specs=[
            pl.BlockSpec((RPB, ROW), lambda g, u: (g * n_ub + u, 0)),
            pl.BlockSpec((RPB, ROW), lambda g, u: (g * n_ub + u, 0)),
            pl.BlockSpec((RPB, ROW), lambda g, u: (g * n_ub + u, 0)),
            pl.BlockSpec((2 * B, ROW), lambda g, u: (0, 0)),
            pl.BlockSpec((ROW, ROW), lambda g, u: (0, 0)),
            pl.BlockSpec((ROW, ROW), lambda g, u: (0, 0)),
            pl.BlockSpec((1, ROW), lambda g, u: (0, 0)),
            pl.BlockSpec((1, ROW), lambda g, u: (0, 0)),
            pl.BlockSpec((ROW, ROW), lambda g, u: (0, 0)),
            pl.BlockSpec((ROW, ROW), lambda g, u: (0, 0)),
        ],
        out_specs=pl.BlockSpec((B, ROW), lambda g, u: (0, 0)),
        out_shape=jax.ShapeDtypeStruct((B, ROW), jnp.float32),
        scratch_shapes=[
            pltpu.VMEM((B, ROW), jnp.float32),
            pltpu.VMEM((B, ROW), jnp.float32),
        ],
    )(gh, gr, gt, gum, Wh, Wr, b1t, w2t, Bones, R)


def kernel(users, movies, user_neighbors, movie_neighbors, input_ids,
           attention_mask, node_emb, relation_emb, att_W1, att_b1, att_W2,
           att_b2, Wu1, bu1, Wu2, bu2, Wv1, bv1, Wv2, bv2):
    del input_ids, attention_mask
    del Wu1, bu1, Wu2, bu2, Wv1, bv1, Wv2, bv2
    del att_b2

    B = users.shape[0]
    NM = user_neighbors.shape[3]
    NL = user_neighbors.shape[1]
    G = 2 * NL

    node_flat = node_emb.reshape(node_emb.shape[0], ROW)
    rel_flat = relation_emb.reshape(relation_emb.shape[0], ROW)
    idx_h = jnp.concatenate(
        [user_neighbors[0].reshape(-1), movie_neighbors[0].reshape(-1)])
    idx_r = jnp.concatenate(
        [user_neighbors[1].reshape(-1), movie_neighbors[1].reshape(-1)])
    idx_t = jnp.concatenate(
        [user_neighbors[2].reshape(-1), movie_neighbors[2].reshape(-1)])
    idx_um = jnp.concatenate([users, movies])

    eye4 = jnp.eye(NF, dtype=jnp.float32)
    Wh = jnp.kron(eye4, att_W1[:DIM, :])
    Wr = jnp.kron(eye4, att_W1[DIM:, :])
    b1t = jnp.tile(att_b1, NF)[None, :]
    w2t = jnp.tile(att_W2[:, 0], NF)[None, :]
    Bones = jnp.kron(eye4, jnp.ones((DIM, DIM), jnp.float32))
    R = jnp.pad(jnp.kron(jnp.ones((NF, 1), jnp.float32),
                         jnp.eye(DIM, dtype=jnp.float32)),
                ((0, 0), (0, ROW - DIM)))

    gh, gr, gt, gum = _sc_gather(idx_h, idx_r, idx_t, idx_um,
                                 node_flat, rel_flat)
    out = _tc_compute(gh, gr, gt, gum, Wh, Wr, b1t, w2t, Bones, R, G, B, NM)
    return out[:, 0]


# speedup vs baseline: 63.2636x; 1.0145x over previous
import functools

import jax
import jax.numpy as jnp
from jax import lax
from jax.experimental import pallas as pl
from jax.experimental.pallas import tpu as pltpu
from jax.experimental.pallas import tpu_sc as plsc

NC, NS = 2, 16
NW = NC * NS
DIM = 32
NF = 4
ROW = NF * DIM
CHUNK = 128


def _sc_gather(idx_h, idx_r, idx_t, idx_b, node_flat, rel_flat):
    NB = idx_h.shape[0]
    per_w = NB // NW
    n_chunks = per_w // CHUNK
    BASE = idx_b.shape[0]
    b_per_w = BASE // NW

    mesh = plsc.VectorSubcoreMesh(core_axis_name="c", subcore_axis_name="s")

    idx_h2 = idx_h.reshape(-1, CHUNK)
    idx_r2 = idx_r.reshape(-1, CHUNK)
    idx_t2 = idx_t.reshape(-1, CHUNK)
    idx_b2 = idx_b.reshape(NW, b_per_w)

    @functools.partial(
        pl.kernel,
        out_type=(
            jax.ShapeDtypeStruct((NB, ROW), jnp.float32),
            jax.ShapeDtypeStruct((NB, ROW), jnp.float32),
            jax.ShapeDtypeStruct((NB, ROW), jnp.float32),
            jax.ShapeDtypeStruct((BASE, ROW), jnp.float32),
        ),
        mesh=mesh,
        scratch_types=[
            pltpu.VMEM((n_chunks, CHUNK), jnp.int32),
            pltpu.VMEM((CHUNK, ROW), jnp.float32),
            pltpu.VMEM((CHUNK, ROW), jnp.float32),
            pltpu.VMEM((b_per_w,), jnp.int32),
            pltpu.VMEM((b_per_w, ROW), jnp.float32),
            pltpu.SemaphoreType.DMA,
            pltpu.SemaphoreType.DMA,
            pltpu.SemaphoreType.DMA,
            pltpu.SemaphoreType.DMA,
        ],
    )
    def gather_kernel(idx_h_hbm, idx_r_hbm, idx_t_hbm, idx_b_hbm,
                      node_hbm, rel_hbm,
                      out_h, out_r, out_t, out_b,
                      idx_all, rows0, rows1, idx_b_v, rows_b_v,
                      gsem0, gsem1, osem0, osem1):
        wid = lax.axis_index("s") * NC + lax.axis_index("c")
        base = wid * per_w
        rows = (rows0, rows1)
        gsem = (gsem0, gsem1)
        osem = (osem0, osem1)

        def run(idx_hbm, tab_hbm, out_hbm):
            pltpu.sync_copy(idx_hbm.at[pl.ds(wid * n_chunks, n_chunks)],
                            idx_all)
            for b in range(2):
                pltpu.async_copy(tab_hbm.at[idx_all.at[b]], rows[b], gsem[b])

            def body(j, carry):
                for b in range(2):
                    i = 2 * j + b
                    off = pl.multiple_of(base + i * CHUNK, CHUNK)
                    pltpu.make_async_copy(tab_hbm.at[idx_all.at[i]],
                                          rows[b], gsem[b]).wait()
                    pltpu.async_copy(rows[b], out_hbm.at[pl.ds(off, CHUNK)],
                                     osem[b])

                    @pl.when(i + 2 < n_chunks)
                    def _next():
                        pltpu.make_async_copy(
                            rows[b], out_hbm.at[pl.ds(off, CHUNK)],
                            osem[b]).wait()
                        pltpu.async_copy(tab_hbm.at[idx_all.at[i + 2]],
                                         rows[b], gsem[b])
                return carry
            lax.fori_loop(0, n_chunks // 2, body, 0)
            for b in range(2):
                pltpu.make_async_copy(rows[b], out_hbm.at[pl.ds(base, CHUNK)],
                                      osem[b]).wait()

        run(idx_h_hbm, node_hbm, out_h)
        run(idx_t_hbm, node_hbm, out_t)
        run(idx_r_hbm, rel_hbm, out_r)

        bb = pl.multiple_of(wid * b_per_w, 8)
        pltpu.sync_copy(idx_b_hbm.at[wid], idx_b_v)
        pltpu.async_copy(node_hbm.at[idx_b_v], rows_b_v, gsem0).wait()
        pltpu.sync_copy(rows_b_v, out_b.at[pl.ds(bb, b_per_w)])

    return gather_kernel(idx_h2, idx_r2, idx_t2, idx_b2, node_flat, rel_flat)


def _tc_side(gh, gr, gt, gb, Wh, Wr, b1t, w2t, Bones, R, G, B, NM,
             e_other=None):
    UB = 128
    RPB = UB * NM
    n_ub = B // UB
    n_steps = G * n_ub
    emit_scores = e_other is not None

    def body(*refs):
        if emit_scores:
            (gh_ref, gr_ref, gt_ref, gb_ref, Wh_ref, Wr_ref, b1_ref, w2_ref,
             Bo_ref, R_ref, eo_ref, out_ref, acc) = refs
        else:
            (gh_ref, gr_ref, gt_ref, gb_ref, Wh_ref, Wr_ref, b1_ref, w2_ref,
             Bo_ref, R_ref, out_ref, acc) = refs
        g = pl.program_id(0)
        u = pl.program_id(1)
        step = g * n_ub + u

        @pl.when(step == 0)
        def _init():
            acc[...] = gb_ref[...]

        H = (jnp.dot(gh_ref[...], Wh_ref[...],
                     preferred_element_type=jnp.float32)
             + jnp.dot(gr_ref[...], Wr_ref[...],
                       preferred_element_type=jnp.float32))
        H = jnp.maximum(H + b1_ref[...], 0.0)
        S = H * w2_ref[...]
        attb = jnp.dot(S, Bo_ref[...], preferred_element_type=jnp.float32)
        att3 = attb.reshape(UB, NM, ROW)
        mx = jnp.max(att3, axis=1, keepdims=True)
        eb = jnp.exp(att3 - mx).reshape(RPB, ROW)
        numer = (eb * gt_ref[...]).reshape(UB, NM, ROW).sum(axis=1)
        denom = eb.reshape(UB, NM, ROW).sum(axis=1)
        gout = numer / denom

        rowbase = u * UB
        acc[pl.ds(rowbase, UB), :] += gout

        @pl.when(step == n_steps - 1)
        def _final():
            if emit_scores:
                evf = jnp.dot(acc[...], R_ref[...],
                              preferred_element_type=jnp.float32,
                              precision=lax.Precision.HIGHEST)
                euf = jnp.dot(eo_ref[...], R_ref[...],
                              preferred_element_type=jnp.float32,
                              precision=lax.Precision.HIGHEST)
                s = jnp.sum(euf * evf, axis=1, keepdims=True)
                out_ref[...] = jnp.broadcast_to(jax.nn.sigmoid(s), (B, ROW))
            else:
                out_ref[...] = acc[...]

    big = pl.BlockSpec((RPB, ROW), lambda g, u: (g * n_ub + u, 0))
    whole = lambda shape: pl.BlockSpec(shape, lambda g, u: (0, 0))
    in_specs = [
        big, big, big,
        whole((B, ROW)),
        whole((ROW, ROW)), whole((ROW, ROW)),
        whole((1, ROW)), whole((1, ROW)),
        whole((ROW, ROW)), whole((ROW, ROW)),
    ]
    args = [gh, gr, gt, gb, Wh, Wr, b1t, w2t, Bones, R]
    if emit_scores:
        in_specs.append(whole((B, ROW)))
        args.append(e_other)

    return pl.pallas_call(
        body,
        grid=(G, n_ub),
        in_specs=in_specs,
        out_specs=pl.BlockSpec((B, ROW), lambda g, u: (0, 0)),
        out_shape=jax.ShapeDtypeStruct((B, ROW), jnp.float32),
        scratch_shapes=[pltpu.VMEM((B, ROW), jnp.float32)],
    )(*args)


def kernel(users, movies, user_neighbors, movie_neighbors, input_ids,
           attention_mask, node_emb, relation_emb, att_W1, att_b1, att_W2,
           att_b2, Wu1, bu1, Wu2, bu2, Wv1, bv1, Wv2, bv2):
    del input_ids, attention_mask
    del Wu1, bu1, Wu2, bu2, Wv1, bv1, Wv2, bv2
    del att_b2

    B = users.shape[0]
    NM = user_neighbors.shape[3]
    NL = user_neighbors.shape[1]

    node_flat = node_emb.reshape(node_emb.shape[0], ROW)
    rel_flat = relation_emb.reshape(relation_emb.shape[0], ROW)

    eye4 = jnp.eye(NF, dtype=jnp.float32)
    Wh = jnp.kron(eye4, att_W1[:DIM, :])
    Wr = jnp.kron(eye4, att_W1[DIM:, :])
    b1t = jnp.tile(att_b1, NF)[None, :]
    w2t = jnp.tile(att_W2[:, 0], NF)[None, :]
    Bones = jnp.kron(eye4, jnp.ones((DIM, DIM), jnp.float32))
    R = jnp.pad(jnp.kron(jnp.ones((NF, 1), jnp.float32),
                         jnp.eye(DIM, dtype=jnp.float32)),
                ((0, 0), (0, ROW - DIM)))

    ghu, gru, gtu, gbu = _sc_gather(
        user_neighbors[0].reshape(-1), user_neighbors[1].reshape(-1),
        user_neighbors[2].reshape(-1), users, node_flat, rel_flat)
    ghm, grm, gtm, gbm = _sc_gather(
        movie_neighbors[0].reshape(-1), movie_neighbors[1].reshape(-1),
        movie_neighbors[2].reshape(-1), movies, node_flat, rel_flat)

    e_u = _tc_side(ghu, gru, gtu, gbu, Wh, Wr, b1t, w2t, Bones, R, NL, B, NM)
    out = _tc_side(ghm, grm, gtm, gbm, Wh, Wr, b1t, w2t, Bones, R, NL, B, NM,
                   e_other=e_u)
    return out[:, 0]


# speedup vs baseline: 63.6495x; 1.0061x over previous
import functools

import jax
import jax.numpy as jnp
from jax import lax
from jax.experimental import pallas as pl
from jax.experimental.pallas import tpu as pltpu
from jax.experimental.pallas import tpu_sc as plsc

NC, NS = 2, 16
NW = NC * NS
DIM = 32
NF = 4
ROW = NF * DIM
CHUNK = 128


def _sc_gather(idx_h, idx_r, idx_t, idx_b, node_flat, rel_flat):
    NB = idx_h.shape[0]
    per_w = NB // NW
    n_chunks = per_w // CHUNK
    BASE = idx_b.shape[0]
    b_per_w = BASE // NW

    mesh = plsc.VectorSubcoreMesh(core_axis_name="c", subcore_axis_name="s")

    idx_h2 = idx_h.reshape(-1, CHUNK)
    idx_r2 = idx_r.reshape(-1, CHUNK)
    idx_t2 = idx_t.reshape(-1, CHUNK)
    idx_b2 = idx_b.reshape(NW, b_per_w)

    @functools.partial(
        pl.kernel,
        out_type=(
            jax.ShapeDtypeStruct((NB, ROW), jnp.float32),
            jax.ShapeDtypeStruct((NB, ROW), jnp.float32),
            jax.ShapeDtypeStruct((NB, ROW), jnp.float32),
            jax.ShapeDtypeStruct((BASE, ROW), jnp.float32),
        ),
        mesh=mesh,
        scratch_types=[
            pltpu.VMEM((n_chunks, CHUNK), jnp.int32),
            pltpu.VMEM((CHUNK, ROW), jnp.float32),
            pltpu.VMEM((CHUNK, ROW), jnp.float32),
            pltpu.VMEM((b_per_w,), jnp.int32),
            pltpu.VMEM((b_per_w, ROW), jnp.float32),
            pltpu.SemaphoreType.DMA,
            pltpu.SemaphoreType.DMA,
            pltpu.SemaphoreType.DMA,
            pltpu.SemaphoreType.DMA,
        ],
    )
    def gather_kernel(idx_h_hbm, idx_r_hbm, idx_t_hbm, idx_b_hbm,
                      node_hbm, rel_hbm,
                      out_h, out_r, out_t, out_b,
                      idx_all, rows0, rows1, idx_b_v, rows_b_v,
                      gsem0, gsem1, osem0, osem1):
        wid = lax.axis_index("s") * NC + lax.axis_index("c")
        base = wid * per_w
        rows = (rows0, rows1)
        gsem = (gsem0, gsem1)
        osem = (osem0, osem1)

        def run(idx_hbm, tab_hbm, out_hbm):
            pltpu.sync_copy(idx_hbm.at[pl.ds(wid * n_chunks, n_chunks)],
                            idx_all)
            for b in range(2):
                pltpu.async_copy(tab_hbm.at[idx_all.at[b]], rows[b], gsem[b])

            def body(j, carry):
                for b in range(2):
                    i = 2 * j + b
                    off = pl.multiple_of(base + i * CHUNK, CHUNK)
                    pltpu.make_async_copy(tab_hbm.at[idx_all.at[i]],
                                          rows[b], gsem[b]).wait()
                    pltpu.async_copy(rows[b], out_hbm.at[pl.ds(off, CHUNK)],
                                     osem[b])

                    @pl.when(i + 2 < n_chunks)
                    def _next():
                        pltpu.make_async_copy(
                            rows[b], out_hbm.at[pl.ds(off, CHUNK)],
                            osem[b]).wait()
                        pltpu.async_copy(tab_hbm.at[idx_all.at[i + 2]],
                                         rows[b], gsem[b])
                return carry
            lax.fori_loop(0, n_chunks // 2, body, 0)
            for b in range(2):
                pltpu.make_async_copy(rows[b], out_hbm.at[pl.ds(base, CHUNK)],
                                      osem[b]).wait()

        run(idx_h_hbm, node_hbm, out_h)
        run(idx_t_hbm, node_hbm, out_t)
        run(idx_r_hbm, rel_hbm, out_r)

        bb = pl.multiple_of(wid * b_per_w, 8)
        pltpu.sync_copy(idx_b_hbm.at[wid], idx_b_v)
        pltpu.async_copy(node_hbm.at[idx_b_v], rows_b_v, gsem0).wait()
        pltpu.sync_copy(rows_b_v, out_b.at[pl.ds(bb, b_per_w)])

    return gather_kernel(idx_h2, idx_r2, idx_t2, idx_b2, node_flat, rel_flat)


def _tc_side(gh, gr, gt, gb, Wh, Wr, b1t, w2t, Bones, R, G, B, NM,
             e_other=None):
    UB = 256
    RPB = UB * NM
    n_ub = B // UB
    n_steps = G * n_ub
    emit_scores = e_other is not None

    def body(*refs):
        if emit_scores:
            (gh_ref, gr_ref, gt_ref, gb_ref, Wh_ref, Wr_ref, b1_ref, w2_ref,
             Bo_ref, R_ref, eo_ref, out_ref, acc) = refs
        else:
            (gh_ref, gr_ref, gt_ref, gb_ref, Wh_ref, Wr_ref, b1_ref, w2_ref,
             Bo_ref, R_ref, out_ref, acc) = refs
        g = pl.program_id(0)
        u = pl.program_id(1)
        step = g * n_ub + u

        @pl.when(step == 0)
        def _init():
            acc[...] = gb_ref[...]

        H = (jnp.dot(gh_ref[...], Wh_ref[...],
                     preferred_element_type=jnp.float32)
             + jnp.dot(gr_ref[...], Wr_ref[...],
                       preferred_element_type=jnp.float32))
        H = jnp.maximum(H + b1_ref[...], 0.0)
        S = H * w2_ref[...]
        attb = jnp.dot(S, Bo_ref[...], preferred_element_type=jnp.float32)
        att3 = attb.reshape(UB, NM, ROW)
        mx = jnp.max(att3, axis=1, keepdims=True)
        eb = jnp.exp(att3 - mx).reshape(RPB, ROW)
        numer = (eb * gt_ref[...]).reshape(UB, NM, ROW).sum(axis=1)
        denom = eb.reshape(UB, NM, ROW).sum(axis=1)
        gout = numer / denom

        rowbase = u * UB
        acc[pl.ds(rowbase, UB), :] += gout

        @pl.when(step == n_steps - 1)
        def _final():
            if emit_scores:
                evf = jnp.dot(acc[...], R_ref[...],
                              preferred_element_type=jnp.float32,
                              precision=lax.Precision.HIGHEST)
                euf = jnp.dot(eo_ref[...], R_ref[...],
                              preferred_element_type=jnp.float32,
                              precision=lax.Precision.HIGHEST)
                s = jnp.sum(euf * evf, axis=1, keepdims=True)
                out_ref[...] = jnp.broadcast_to(jax.nn.sigmoid(s), (B, ROW))
            else:
                out_ref[...] = acc[...]

    big = pl.BlockSpec((RPB, ROW), lambda g, u: (g * n_ub + u, 0))
    whole = lambda shape: pl.BlockSpec(shape, lambda g, u: (0, 0))
    in_specs = [
        big, big, big,
        whole((B, ROW)),
        whole((ROW, ROW)), whole((ROW, ROW)),
        whole((1, ROW)), whole((1, ROW)),
        whole((ROW, ROW)), whole((ROW, ROW)),
    ]
    args = [gh, gr, gt, gb, Wh, Wr, b1t, w2t, Bones, R]
    if emit_scores:
        in_specs.append(whole((B, ROW)))
        args.append(e_other)

    return pl.pallas_call(
        body,
        grid=(G, n_ub),
        in_specs=in_specs,
        out_specs=pl.BlockSpec((B, ROW), lambda g, u: (0, 0)),
        out_shape=jax.ShapeDtypeStruct((B, ROW), jnp.float32),
        scratch_shapes=[pltpu.VMEM((B, ROW), jnp.float32)],
    )(*args)


def kernel(users, movies, user_neighbors, movie_neighbors, input_ids,
           attention_mask, node_emb, relation_emb, att_W1, att_b1, att_W2,
           att_b2, Wu1, bu1, Wu2, bu2, Wv1, bv1, Wv2, bv2):
    del input_ids, attention_mask
    del Wu1, bu1, Wu2, bu2, Wv1, bv1, Wv2, bv2
    del att_b2

    B = users.shape[0]
    NM = user_neighbors.shape[3]
    NL = user_neighbors.shape[1]

    node_flat = node_emb.reshape(node_emb.shape[0], ROW)
    rel_flat = relation_emb.reshape(relation_emb.shape[0], ROW)

    eye4 = jnp.eye(NF, dtype=jnp.float32)
    Wh = jnp.kron(eye4, att_W1[:DIM, :])
    Wr = jnp.kron(eye4, att_W1[DIM:, :])
    b1t = jnp.tile(att_b1, NF)[None, :]
    w2t = jnp.tile(att_W2[:, 0], NF)[None, :]
    Bones = jnp.kron(eye4, jnp.ones((DIM, DIM), jnp.float32))
    R = jnp.pad(jnp.kron(jnp.ones((NF, 1), jnp.float32),
                         jnp.eye(DIM, dtype=jnp.float32)),
                ((0, 0), (0, ROW - DIM)))

    ghu, gru, gtu, gbu = _sc_gather(
        user_neighbors[0].reshape(-1), user_neighbors[1].reshape(-1),
        user_neighbors[2].reshape(-1), users, node_flat, rel_flat)
    ghm, grm, gtm, gbm = _sc_gather(
        movie_neighbors[0].reshape(-1), movie_neighbors[1].reshape(-1),
        movie_neighbors[2].reshape(-1), movies, node_flat, rel_flat)

    e_u = _tc_side(ghu, gru, gtu, gbu, Wh, Wr, b1t, w2t, Bones, R, NL, B, NM)
    out = _tc_side(ghm, grm, gtm, gbm, Wh, Wr, b1t, w2t, Bones, R, NL, B, NM,
                   e_other=e_u)
    return out[:, 0]


# speedup vs baseline: 65.2712x; 1.0255x over previous
import functools

import jax
import jax.numpy as jnp
from jax import lax
from jax.experimental import pallas as pl
from jax.experimental.pallas import tpu as pltpu
from jax.experimental.pallas import tpu_sc as plsc

NC, NS = 2, 16
NW = NC * NS
DIM = 32
NF = 4
ROW = NF * DIM
CHUNK = 128


def _pipelined_gather(idx_hbm, tab_hbm, out_hbm, idx_all, rows, gsem, osem,
                      wid, per_w, n_chunks):
    base = wid * per_w
    pltpu.sync_copy(idx_hbm.at[pl.ds(wid * n_chunks, n_chunks)], idx_all)
    for b in range(2):
        pltpu.async_copy(tab_hbm.at[idx_all.at[b]], rows[b], gsem[b])

    def body(j, carry):
        for b in range(2):
            i = 2 * j + b
            off = pl.multiple_of(base + i * CHUNK, CHUNK)
            pltpu.make_async_copy(tab_hbm.at[idx_all.at[i]],
                                  rows[b], gsem[b]).wait()
            pltpu.async_copy(rows[b], out_hbm.at[pl.ds(off, CHUNK)], osem[b])

            @pl.when(i + 2 < n_chunks)
            def _next():
                pltpu.make_async_copy(rows[b], out_hbm.at[pl.ds(off, CHUNK)],
                                      osem[b]).wait()
                pltpu.async_copy(tab_hbm.at[idx_all.at[i + 2]],
                                 rows[b], gsem[b])
        return carry
    lax.fori_loop(0, n_chunks // 2, body, 0)
    for b in range(2):
        pltpu.make_async_copy(rows[b], out_hbm.at[pl.ds(wid * per_w, CHUNK)],
                              osem[b]).wait()


def _sc_gather_node(idx_h, idx_t, idx_b, node_flat):
    NB = idx_h.shape[0]
    per_w = NB // NW
    n_chunks = per_w // CHUNK
    BASE = idx_b.shape[0]
    b_per_w = BASE // NW

    mesh = plsc.VectorSubcoreMesh(core_axis_name="c", subcore_axis_name="s")

    idx_h2 = idx_h.reshape(-1, CHUNK)
    idx_t2 = idx_t.reshape(-1, CHUNK)
    idx_b2 = idx_b.reshape(NW, b_per_w)

    @functools.partial(
        pl.kernel,
        out_type=(
            jax.ShapeDtypeStruct((NB, ROW), jnp.float32),
            jax.ShapeDtypeStruct((NB, ROW), jnp.float32),
            jax.ShapeDtypeStruct((BASE, ROW), jnp.float32),
        ),
        mesh=mesh,
        scratch_types=[
            pltpu.VMEM((n_chunks, CHUNK), jnp.int32),
            pltpu.VMEM((CHUNK, ROW), jnp.float32),
            pltpu.VMEM((CHUNK, ROW), jnp.float32),
            pltpu.VMEM((b_per_w,), jnp.int32),
            pltpu.VMEM((b_per_w, ROW), jnp.float32),
            pltpu.SemaphoreType.DMA,
            pltpu.SemaphoreType.DMA,
            pltpu.SemaphoreType.DMA,
            pltpu.SemaphoreType.DMA,
        ],
    )
    def gather_kernel(idx_h_hbm, idx_t_hbm, idx_b_hbm, node_hbm,
                      out_h, out_t, out_b,
                      idx_all, rows0, rows1, idx_b_v, rows_b_v,
                      gsem0, gsem1, osem0, osem1):
        wid = lax.axis_index("s") * NC + lax.axis_index("c")
        rows = (rows0, rows1)
        gsem = (gsem0, gsem1)
        osem = (osem0, osem1)
        _pipelined_gather(idx_h_hbm, node_hbm, out_h, idx_all, rows,
                          gsem, osem, wid, per_w, n_chunks)
        _pipelined_gather(idx_t_hbm, node_hbm, out_t, idx_all, rows,
                          gsem, osem, wid, per_w, n_chunks)
        bb = pl.multiple_of(wid * b_per_w, 8)
        pltpu.sync_copy(idx_b_hbm.at[wid], idx_b_v)
        pltpu.async_copy(node_hbm.at[idx_b_v], rows_b_v, gsem0).wait()
        pltpu.sync_copy(rows_b_v, out_b.at[pl.ds(bb, b_per_w)])

    return gather_kernel(idx_h2, idx_t2, idx_b2, node_flat)


def _sc_gather_rel(idx_r, rel_flat):
    NB = idx_r.shape[0]
    per_w = NB // NW
    n_chunks = per_w // CHUNK

    mesh = plsc.VectorSubcoreMesh(core_axis_name="c", subcore_axis_name="s")
    idx_r2 = idx_r.reshape(-1, CHUNK)

    @functools.partial(
        pl.kernel,
        out_type=jax.ShapeDtypeStruct((NB, ROW), jnp.float32),
        mesh=mesh,
        scratch_types=[
            pltpu.VMEM((n_chunks, CHUNK), jnp.int32),
            pltpu.VMEM((CHUNK, ROW), jnp.float32),
            pltpu.VMEM((CHUNK, ROW), jnp.float32),
            pltpu.SemaphoreType.DMA,
            pltpu.SemaphoreType.DMA,
            pltpu.SemaphoreType.DMA,
            pltpu.SemaphoreType.DMA,
        ],
    )
    def gather_kernel(idx_r_hbm, rel_hbm, out_r,
                      idx_all, rows0, rows1, gsem0, gsem1, osem0, osem1):
        wid = lax.axis_index("s") * NC + lax.axis_index("c")
        _pipelined_gather(idx_r_hbm, rel_hbm, out_r, idx_all,
                          (rows0, rows1), (gsem0, gsem1), (osem0, osem1),
                          wid, per_w, n_chunks)

    return gather_kernel(idx_r2, rel_flat)


def _tc_side(gh, gr, gt, gb, Wh, Wr, b1t, w2t, Bones, R, G, B, NM,
             e_other=None):
    UB = 256
    RPB = UB * NM
    n_ub = B // UB
    n_steps = G * n_ub
    emit_scores = e_other is not None

    def body(*refs):
        if emit_scores:
            (gh_ref, gr_ref, gt_ref, gb_ref, Wh_ref, Wr_ref, b1_ref, w2_ref,
             Bo_ref, R_ref, eo_ref, out_ref, acc) = refs
        else:
            (gh_ref, gr_ref, gt_ref, gb_ref, Wh_ref, Wr_ref, b1_ref, w2_ref,
             Bo_ref, R_ref, out_ref, acc) = refs
        g = pl.program_id(0)
        u = pl.program_id(1)
        step = g * n_ub + u

        @pl.when(step == 0)
        def _init():
            acc[...] = gb_ref[...]

        H = (jnp.dot(gh_ref[...], Wh_ref[...],
                     preferred_element_type=jnp.float32)
             + jnp.dot(gr_ref[...], Wr_ref[...],
                       preferred_element_type=jnp.float32))
        H = jnp.maximum(H + b1_ref[...], 0.0)
        S = H * w2_ref[...]
        attb = jnp.dot(S, Bo_ref[...], preferred_element_type=jnp.float32)
        att3 = attb.reshape(UB, NM, ROW)
        mx = jnp.max(att3, axis=1, keepdims=True)
        eb = jnp.exp(att3 - mx).reshape(RPB, ROW)
        numer = (eb * gt_ref[...]).reshape(UB, NM, ROW).sum(axis=1)
        denom = eb.reshape(UB, NM, ROW).sum(axis=1)
        gout = numer / denom

        rowbase = u * UB
        acc[pl.ds(rowbase, UB), :] += gout

        @pl.when(step == n_steps - 1)
        def _final():
            if emit_scores:
                evf = jnp.dot(acc[...], R_ref[...],
                              preferred_element_type=jnp.float32,
                              precision=lax.Precision.HIGHEST)
                euf = jnp.dot(eo_ref[...], R_ref[...],
                              preferred_element_type=jnp.float32,
                              precision=lax.Precision.HIGHEST)
                s = jnp.sum(euf * evf, axis=1, keepdims=True)
                out_ref[...] = jnp.broadcast_to(jax.nn.sigmoid(s), (B, ROW))
            else:
                out_ref[...] = acc[...]

    big = pl.BlockSpec((RPB, ROW), lambda g, u: (g * n_ub + u, 0))
    whole = lambda shape: pl.BlockSpec(shape, lambda g, u: (0, 0))
    in_specs = [
        big, big, big,
        whole((B, ROW)),
        whole((ROW, ROW)), whole((ROW, ROW)),
        whole((1, ROW)), whole((1, ROW)),
        whole((ROW, ROW)), whole((ROW, ROW)),
    ]
    args = [gh, gr, gt, gb, Wh, Wr, b1t, w2t, Bones, R]
    if emit_scores:
        in_specs.append(whole((B, ROW)))
        args.append(e_other)

    return pl.pallas_call(
        body,
        grid=(G, n_ub),
        in_specs=in_specs,
        out_specs=pl.BlockSpec((B, ROW), lambda g, u: (0, 0)),
        out_shape=jax.ShapeDtypeStruct((B, ROW), jnp.float32),
        scratch_shapes=[pltpu.VMEM((B, ROW), jnp.float32)],
    )(*args)


def kernel(users, movies, user_neighbors, movie_neighbors, input_ids,
           attention_mask, node_emb, relation_emb, att_W1, att_b1, att_W2,
           att_b2, Wu1, bu1, Wu2, bu2, Wv1, bv1, Wv2, bv2):
    del input_ids, attention_mask
    del Wu1, bu1, Wu2, bu2, Wv1, bv1, Wv2, bv2
    del att_b2

    B = users.shape[0]
    NM = user_neighbors.shape[3]
    NL = user_neighbors.shape[1]

    node_flat = node_emb.reshape(node_emb.shape[0], ROW)
    rel_flat = relation_emb.reshape(relation_emb.shape[0], ROW)

    eye4 = jnp.eye(NF, dtype=jnp.float32)
    Wh = jnp.kron(eye4, att_W1[:DIM, :])
    Wr = jnp.kron(eye4, att_W1[DIM:, :])
    b1t = jnp.tile(att_b1, NF)[None, :]
    w2t = jnp.tile(att_W2[:, 0], NF)[None, :]
    Bones = jnp.kron(eye4, jnp.ones((DIM, DIM), jnp.float32))
    R = jnp.pad(jnp.kron(jnp.ones((NF, 1), jnp.float32),
                         jnp.eye(DIM, dtype=jnp.float32)),
                ((0, 0), (0, ROW - DIM)))

    ghu, gtu, gbu = _sc_gather_node(
        user_neighbors[0].reshape(-1), user_neighbors[2].reshape(-1),
        users, node_flat)
    gru = _sc_gather_rel(user_neighbors[1].reshape(-1), rel_flat)
    ghm, gtm, gbm = _sc_gather_node(
        movie_neighbors[0].reshape(-1), movie_neighbors[2].reshape(-1),
        movies, node_flat)
    grm = _sc_gather_rel(movie_neighbors[1].reshape(-1), rel_flat)

    e_u = _tc_side(ghu, gru, gtu, gbu, Wh, Wr, b1t, w2t, Bones, R, NL, B, NM)
    out = _tc_side(ghm, grm, gtm, gbm, Wh, Wr, b1t, w2t, Bones, R, NL, B, NM,
                   e_other=e_u)
    return out[:, 0]


# speedup vs baseline: 65.3567x; 1.0013x over previous
import functools

import jax
import jax.numpy as jnp
from jax import lax
from jax.experimental import pallas as pl
from jax.experimental.pallas import tpu as pltpu
from jax.experimental.pallas import tpu_sc as plsc

NC, NS = 2, 16
NW = NC * NS
DIM = 32
NF = 4
ROW = NF * DIM
CHUNK = 128


def _pipelined_gather(idx_hbm, tab_hbm, out_hbm, idx_all, rows, gsem, osem,
                      wid, per_w, n_chunks):
    base = wid * per_w
    pltpu.sync_copy(idx_hbm.at[pl.ds(wid * n_chunks, n_chunks)], idx_all)
    for b in range(2):
        pltpu.async_copy(tab_hbm.at[idx_all.at[b]], rows[b], gsem[b])

    def body(j, carry):
        for b in range(2):
            i = 2 * j + b
            off = pl.multiple_of(base + i * CHUNK, CHUNK)
            pltpu.make_async_copy(tab_hbm.at[idx_all.at[i]],
                                  rows[b], gsem[b]).wait()
            pltpu.async_copy(rows[b], out_hbm.at[pl.ds(off, CHUNK)], osem[b])

            @pl.when(i + 2 < n_chunks)
            def _next():
                pltpu.make_async_copy(rows[b], out_hbm.at[pl.ds(off, CHUNK)],
                                      osem[b]).wait()
                pltpu.async_copy(tab_hbm.at[idx_all.at[i + 2]],
                                 rows[b], gsem[b])
        return carry
    lax.fori_loop(0, n_chunks // 2, body, 0)
    for b in range(2):
        pltpu.make_async_copy(rows[b], out_hbm.at[pl.ds(wid * per_w, CHUNK)],
                              osem[b]).wait()


def _sc_gather_node(idx_h, idx_t, idx_b, node_flat):
    NB = idx_h.shape[0]
    per_w = NB // NW
    n_chunks = per_w // CHUNK
    BASE = idx_b.shape[0]
    b_per_w = BASE // NW

    mesh = plsc.VectorSubcoreMesh(core_axis_name="c", subcore_axis_name="s")

    idx_h2 = idx_h.reshape(-1, CHUNK)
    idx_t2 = idx_t.reshape(-1, CHUNK)
    idx_b2 = idx_b.reshape(NW, b_per_w)

    @functools.partial(
        pl.kernel,
        out_type=(
            jax.ShapeDtypeStruct((NB, ROW), jnp.float32),
            jax.ShapeDtypeStruct((NB, ROW), jnp.float32),
            jax.ShapeDtypeStruct((BASE, ROW), jnp.float32),
        ),
        mesh=mesh,
        scratch_types=[
            pltpu.VMEM((n_chunks, CHUNK), jnp.int32),
            pltpu.VMEM((CHUNK, ROW), jnp.float32),
            pltpu.VMEM((CHUNK, ROW), jnp.float32),
            pltpu.VMEM((b_per_w,), jnp.int32),
            pltpu.VMEM((b_per_w, ROW), jnp.float32),
            pltpu.SemaphoreType.DMA,
            pltpu.SemaphoreType.DMA,
            pltpu.SemaphoreType.DMA,
            pltpu.SemaphoreType.DMA,
        ],
    )
    def gather_kernel(idx_h_hbm, idx_t_hbm, idx_b_hbm, node_hbm,
                      out_h, out_t, out_b,
                      idx_all, rows0, rows1, idx_b_v, rows_b_v,
                      gsem0, gsem1, osem0, osem1):
        wid = lax.axis_index("s") * NC + lax.axis_index("c")
        rows = (rows0, rows1)
        gsem = (gsem0, gsem1)
        osem = (osem0, osem1)
        _pipelined_gather(idx_h_hbm, node_hbm, out_h, idx_all, rows,
                          gsem, osem, wid, per_w, n_chunks)
        _pipelined_gather(idx_t_hbm, node_hbm, out_t, idx_all, rows,
                          gsem, osem, wid, per_w, n_chunks)
        bb = pl.multiple_of(wid * b_per_w, 8)
        pltpu.sync_copy(idx_b_hbm.at[wid], idx_b_v)
        pltpu.async_copy(node_hbm.at[idx_b_v], rows_b_v, gsem0).wait()
        pltpu.sync_copy(rows_b_v, out_b.at[pl.ds(bb, b_per_w)])

    return gather_kernel(idx_h2, idx_t2, idx_b2, node_flat)


def _sc_gather_rel(idx_r, rel_flat):
    NB = idx_r.shape[0]
    per_w = NB // NW
    n_chunks = per_w // CHUNK

    mesh = plsc.VectorSubcoreMesh(core_axis_name="c", subcore_axis_name="s")
    idx_r2 = idx_r.reshape(-1, CHUNK)

    @functools.partial(
        pl.kernel,
        out_type=jax.ShapeDtypeStruct((NB, ROW), jnp.float32),
        mesh=mesh,
        scratch_types=[
            pltpu.VMEM((n_chunks, CHUNK), jnp.int32),
            pltpu.VMEM((CHUNK, ROW), jnp.float32),
            pltpu.VMEM((CHUNK, ROW), jnp.float32),
            pltpu.SemaphoreType.DMA,
            pltpu.SemaphoreType.DMA,
            pltpu.SemaphoreType.DMA,
            pltpu.SemaphoreType.DMA,
        ],
    )
    def gather_kernel(idx_r_hbm, rel_hbm, out_r,
                      idx_all, rows0, rows1, gsem0, gsem1, osem0, osem1):
        wid = lax.axis_index("s") * NC + lax.axis_index("c")
        _pipelined_gather(idx_r_hbm, rel_hbm, out_r, idx_all,
                          (rows0, rows1), (gsem0, gsem1), (osem0, osem1),
                          wid, per_w, n_chunks)

    return gather_kernel(idx_r2, rel_flat)


def _tc_side(gh, gr, gt, gb, Wh, Wr, b1t, w2t, Bones, R, G, B, NM,
             e_other=None):
    UB = 256
    RPB = UB * NM
    n_ub = B // UB
    n_steps = G * n_ub
    emit_scores = e_other is not None

    def body(*refs):
        if emit_scores:
            (gh_ref, gr_ref, gt_ref, gb_ref, Wh_ref, Wr_ref, b1_ref, w2_ref,
             Bo_ref, R_ref, eo_ref, out_ref, acc) = refs
        else:
            (gh_ref, gr_ref, gt_ref, gb_ref, Wh_ref, Wr_ref, b1_ref, w2_ref,
             Bo_ref, R_ref, out_ref, acc) = refs
        g = pl.program_id(0)
        u = pl.program_id(1)
        step = g * n_ub + u

        @pl.when(step == 0)
        def _init():
            acc[...] = gb_ref[...]

        H = (jnp.dot(gh_ref[...], Wh_ref[...],
                     preferred_element_type=jnp.float32)
             + jnp.dot(gr_ref[...], Wr_ref[...],
                       preferred_element_type=jnp.float32))
        H = jnp.maximum(H + b1_ref[...], 0.0)
        S = H * w2_ref[...]
        attb = jnp.dot(S, Bo_ref[...], preferred_element_type=jnp.float32)
        att3 = attb.reshape(UB, NM, ROW)
        mx = jnp.max(att3, axis=1, keepdims=True)
        eb = jnp.exp(att3 - mx).reshape(RPB, ROW)
        numer = (eb * gt_ref[...]).reshape(UB, NM, ROW).sum(axis=1)
        denom = eb.reshape(UB, NM, ROW).sum(axis=1)
        gout = numer / denom

        rowbase = u * UB
        acc[pl.ds(rowbase, UB), :] += gout

        @pl.when(step == n_steps - 1)
        def _final():
            if emit_scores:
                evf = jnp.dot(acc[...], R_ref[...],
                              preferred_element_type=jnp.float32,
                              precision=lax.Precision.HIGHEST)
                euf = jnp.dot(eo_ref[...], R_ref[...],
                              preferred_element_type=jnp.float32,
                              precision=lax.Precision.HIGHEST)
                s = jnp.sum(euf * evf, axis=1, keepdims=True)
                out_ref[...] = jnp.broadcast_to(jax.nn.sigmoid(s), (B, ROW))
            else:
                out_ref[...] = acc[...]

    big = pl.BlockSpec((RPB, ROW), lambda g, u: (g * n_ub + u, 0))
    whole = lambda shape: pl.BlockSpec(shape, lambda g, u: (0, 0))
    in_specs = [
        big, big, big,
        whole((B, ROW)),
        whole((ROW, ROW)), whole((ROW, ROW)),
        whole((1, ROW)), whole((1, ROW)),
        whole((ROW, ROW)), whole((ROW, ROW)),
    ]
    args = [gh, gr, gt, gb, Wh, Wr, b1t, w2t, Bones, R]
    if emit_scores:
        in_specs.append(whole((B, ROW)))
        args.append(e_other)

    return pl.pallas_call(
        body,
        grid=(G, n_ub),
        in_specs=in_specs,
        out_specs=pl.BlockSpec((B, ROW), lambda g, u: (0, 0)),
        out_shape=jax.ShapeDtypeStruct((B, ROW), jnp.float32),
        scratch_shapes=[pltpu.VMEM((B, ROW), jnp.float32)],
    )(*args)


def kernel(users, movies, user_neighbors, movie_neighbors, input_ids,
           attention_mask, node_emb, relation_emb, att_W1, att_b1, att_W2,
           att_b2, Wu1, bu1, Wu2, bu2, Wv1, bv1, Wv2, bv2):
    del input_ids, attention_mask
    del Wu1, bu1, Wu2, bu2, Wv1, bv1, Wv2, bv2
    del att_b2

    B = users.shape[0]
    NM = user_neighbors.shape[3]
    NL = user_neighbors.shape[1]

    node_flat = node_emb.reshape(node_emb.shape[0], ROW)
    rel_flat = relation_emb.reshape(relation_emb.shape[0], ROW)

    eye4 = jnp.eye(NF, dtype=jnp.float32)
    Wh = jnp.kron(eye4, att_W1[:DIM, :])
    Wr = jnp.kron(eye4, att_W1[DIM:, :])
    b1t = jnp.tile(att_b1, NF)[None, :]
    w2t = jnp.tile(att_W2[:, 0], NF)[None, :]
    Bones = jnp.kron(eye4, jnp.ones((DIM, DIM), jnp.float32))
    R = jnp.pad(jnp.kron(jnp.ones((NF, 1), jnp.float32),
                         jnp.eye(DIM, dtype=jnp.float32)),
                ((0, 0), (0, ROW - DIM)))

    gru = _sc_gather_rel(user_neighbors[1].reshape(-1), rel_flat)
    grm = _sc_gather_rel(movie_neighbors[1].reshape(-1), rel_flat)
    ghu, gtu, gbu = _sc_gather_node(
        user_neighbors[0].reshape(-1), user_neighbors[2].reshape(-1),
        users, node_flat)
    ghm, gtm, gbm = _sc_gather_node(
        movie_neighbors[0].reshape(-1), movie_neighbors[2].reshape(-1),
        movies, node_flat)

    e_u = _tc_side(ghu, gru, gtu, gbu, Wh, Wr, b1t, w2t, Bones, R, NL, B, NM)
    out = _tc_side(ghm, grm, gtm, gbm, Wh, Wr, b1t, w2t, Bones, R, NL, B, NM,
                   e_other=e_u)
    return out[:, 0]
